# Initial kernel scaffold; baseline (speedup 1.0000x reference)
import jax
import jax.numpy as jnp
from jax.experimental import pallas as pl


def kernel(input, embeddings):
    raise NotImplementedError("write your pallas kernel here")


# speedup vs baseline: 1.0223x; 1.0223x over previous
import functools

import jax
import jax.numpy as jnp
from jax import lax
from jax.experimental import pallas as pl
from jax.experimental.pallas import tpu as pltpu
from jax.experimental.pallas import tpu_sc as plsc

_D = 32
_B = 16384 * 50
_NW = 32
_R = _B // _NW
_K = 128
_NCHUNK = _R // _K

_mesh = plsc.VectorSubcoreMesh(core_axis_name="c", subcore_axis_name="s")


@functools.partial(
    pl.kernel,
    out_type=jax.ShapeDtypeStruct((_B, _D), jnp.float32),
    mesh=_mesh,
    scratch_types=[
        pltpu.VMEM((_R,), jnp.int32),
        pltpu.VMEM((_K, _D), jnp.float32),
        pltpu.SemaphoreType.DMA,
    ],
    compiler_params=pltpu.CompilerParams(use_tc_tiling_on_sc=False),
)
def _gather_kernel(idx_hbm, table_hbm, out_hbm, idx_v, rows_v, sem):
    wid = lax.axis_index("s") * 2 + lax.axis_index("c")
    base = wid * _R
    pltpu.sync_copy(idx_hbm.at[pl.ds(base, _R)], idx_v)

    def body(j, carry):
        pltpu.async_copy(
            table_hbm.at[idx_v.at[pl.ds(j * _K, _K)]], rows_v, sem
        ).wait()
        pltpu.sync_copy(rows_v, out_hbm.at[pl.ds(base + j * _K, _K)])
        return carry

    lax.fori_loop(0, _NCHUNK, body, 0)


def kernel(input, embeddings):
    idx = input.reshape(-1).astype(jnp.int32)
    out = _gather_kernel(idx, embeddings)
    return out.reshape(input.shape + (_D,))


# speedup vs baseline: 1.1041x; 1.0800x over previous
import functools

import jax
import jax.numpy as jnp
from jax import lax
from jax.experimental import pallas as pl
from jax.experimental.pallas import tpu as pltpu
from jax.experimental.pallas import tpu_sc as plsc

_D = 32
_B = 16384 * 50
_NW = 32
_R = _B // _NW
_K = 128
_NCHUNK = _R // _K

_mesh = plsc.VectorSubcoreMesh(core_axis_name="c", subcore_axis_name="s")


_NBUF = 4
_NGROUP = _NCHUNK // _NBUF


@functools.partial(
    pl.kernel,
    out_type=jax.ShapeDtypeStruct((_B, _D), jnp.float32),
    mesh=_mesh,
    scratch_types=[
        pltpu.VMEM((_R,), jnp.int32),
        [pltpu.VMEM((_K, _D), jnp.float32) for _ in range(_NBUF)],
        [pltpu.SemaphoreType.DMA for _ in range(_NBUF)],
        [pltpu.SemaphoreType.DMA for _ in range(_NBUF)],
    ],
    compiler_params=pltpu.CompilerParams(use_tc_tiling_on_sc=False),
)
def _gather_kernel(idx_hbm, table_hbm, out_hbm, idx_v, rows, gsem, wsem):
    wid = lax.axis_index("s") * 2 + lax.axis_index("c")
    base = wid * _R
    pltpu.sync_copy(idx_hbm.at[pl.ds(base, _R)], idx_v)

    def gather_start(j, b):
        pltpu.async_copy(
            table_hbm.at[idx_v.at[pl.ds(j * _K, _K)]], rows[b], gsem[b]
        )

    def gather_wait(j, b):
        pltpu.make_async_copy(
            table_hbm.at[idx_v.at[pl.ds(j * _K, _K)]], rows[b], gsem[b]
        ).wait()

    def write_start(j, b):
        pltpu.async_copy(
            rows[b], out_hbm.at[pl.ds(base + j * _K, _K)], wsem[b]
        )

    def write_wait(j, b):
        pltpu.make_async_copy(
            rows[b], out_hbm.at[pl.ds(base + j * _K, _K)], wsem[b]
        ).wait()

    for b in range(_NBUF):
        gather_start(b, b)

    def group(g, carry):
        j0 = g * _NBUF
        for b in range(_NBUF):
            gather_wait(j0 + b, b)
            write_start(j0 + b, b)
        @pl.when(g + 1 < _NGROUP)
        def _():
            for b in range(_NBUF):
                write_wait(j0 + b, b)
                gather_start(j0 + _NBUF + b, b)

        return carry

    lax.fori_loop(0, _NGROUP, group, 0)
    for b in range(_NBUF):
        write_wait(_NCHUNK - _NBUF + b, b)


def kernel(input, embeddings):
    idx = input.reshape(-1).astype(jnp.int32)
    out = _gather_kernel(idx, embeddings)
    return out.reshape(input.shape + (_D,))


# speedup vs baseline: 1.6967x; 1.5368x over previous
import functools

import jax
import jax.numpy as jnp
from jax import lax
from jax.experimental import pallas as pl
from jax.experimental.pallas import tpu as pltpu
from jax.experimental.pallas import tpu_sc as plsc

_D = 32
_NB = 16384
_S = 50
_NW = 32
_RB = _NB // _NW

_NBUF = 4
_NGROUP = _RB // _NBUF


_mesh = plsc.VectorSubcoreMesh(core_axis_name="c", subcore_axis_name="s")


@functools.partial(
    pl.kernel,
    out_type=jax.ShapeDtypeStruct((_NB, _S, _D), jnp.float32),
    mesh=_mesh,
    scratch_types=[
        pltpu.VMEM((_RB, _S), jnp.int32),
        [pltpu.VMEM((_S, _D), jnp.float32) for _ in range(_NBUF)],
        [pltpu.SemaphoreType.DMA for _ in range(_NBUF)],
        [pltpu.SemaphoreType.DMA for _ in range(_NBUF)],
    ],
    compiler_params=pltpu.CompilerParams(use_tc_tiling_on_sc=False),
)
def _gather_kernel(idx_hbm, table_hbm, out_hbm, idx_v, rows, gsem, wsem):
    wid = lax.axis_index("s") * 2 + lax.axis_index("c")
    base = wid * _RB
    pltpu.sync_copy(idx_hbm.at[pl.ds(base, _RB)], idx_v)

    def gather_start(i, b):
        pltpu.async_copy(table_hbm.at[idx_v.at[i]], rows[b], gsem[b])

    def gather_wait(i, b):
        pltpu.make_async_copy(table_hbm.at[idx_v.at[i]], rows[b], gsem[b]).wait()

    def write_start(i, b):
        pltpu.async_copy(rows[b], out_hbm.at[base + i], wsem[b])

    def write_wait(i, b):
        pltpu.make_async_copy(rows[b], out_hbm.at[base + i], wsem[b]).wait()

    for b in range(_NBUF):
        gather_start(b, b)

    def group(g, carry):
        i0 = g * _NBUF
        for b in range(_NBUF):
            gather_wait(i0 + b, b)
            write_start(i0 + b, b)
        @pl.when(g + 1 < _NGROUP)
        def _():
            for b in range(_NBUF):
                write_wait(i0 + b, b)
                gather_start(i0 + _NBUF + b, b)

        return carry

    lax.fori_loop(0, _NGROUP, group, 0)
    for b in range(_NBUF):
        write_wait(_RB - _NBUF + b, b)


def kernel(input, embeddings):
    return _gather_kernel(input.astype(jnp.int32), embeddings)


# speedup vs baseline: 1.6981x; 1.0008x over previous
import functools

import jax
import jax.numpy as jnp
from jax import lax
from jax.experimental import pallas as pl
from jax.experimental.pallas import tpu as pltpu
from jax.experimental.pallas import tpu_sc as plsc

_D = 32
_NB = 16384
_S = 50
_SP = 64
_NW = 32
_RB = _NB // _NW

_mesh = plsc.VectorSubcoreMesh(core_axis_name="c", subcore_axis_name="s")


@functools.partial(
    pl.kernel,
    out_type=jax.ShapeDtypeStruct((_NB * _SP,), jnp.int32),
    mesh=_mesh,
    scratch_types=[
        pltpu.VMEM((_RB, _S), jnp.int32),
        pltpu.VMEM((_RB * _SP,), jnp.int32),
    ],
)
def _prep(idx_hbm, idxp_hbm, vin, vout):
    wid = lax.axis_index("s") * 2 + lax.axis_index("c")
    i0 = wid * _RB
    pltpu.sync_copy(idx_hbm.at[pl.ds(i0, _RB)], vin)

    def row(r, carry):
        for k in (0, 16, 32, 34):
            vout[pl.ds(r * _SP + k, 16)] = vin[r, pl.ds(k, 16)]
        return carry

    lax.fori_loop(0, _RB, row, 0)
    pltpu.sync_copy(vout, idxp_hbm.at[pl.ds(i0 * _SP, _RB * _SP)])


_NBUF = 4
_NGROUP = _RB // _NBUF


@functools.partial(
    pl.kernel,
    out_type=jax.ShapeDtypeStruct((_NB, _S, _D), jnp.float32),
    mesh=_mesh,
    scratch_types=[
        pltpu.VMEM((_RB * _SP,), jnp.int32),
        [pltpu.VMEM((_S, _D), jnp.float32) for _ in range(_NBUF)],
        [pltpu.SemaphoreType.DMA for _ in range(_NBUF)],
        [pltpu.SemaphoreType.DMA for _ in range(_NBUF)],
    ],
    compiler_params=pltpu.CompilerParams(use_tc_tiling_on_sc=False),
)
def _gather(idxp_hbm, table_hbm, out_hbm, idx_v, rows, gsem, wsem):
    wid = lax.axis_index("s") * 2 + lax.axis_index("c")
    i0 = wid * _RB
    pltpu.sync_copy(idxp_hbm.at[pl.ds(i0 * _SP, _RB * _SP)], idx_v)

    def gather_start(i, b):
        pltpu.async_copy(
            table_hbm.at[idx_v.at[pl.ds(i * _SP, _S)]], rows[b], gsem[b]
        )

    def gather_wait(i, b):
        pltpu.make_async_copy(
            table_hbm.at[idx_v.at[pl.ds(i * _SP, _S)]], rows[b], gsem[b]
        ).wait()

    def write_start(i, b):
        pltpu.async_copy(rows[b], out_hbm.at[i0 + i], wsem[b])

    def write_wait(i, b):
        pltpu.make_async_copy(rows[b], out_hbm.at[i0 + i], wsem[b]).wait()

    for b in range(_NBUF):
        gather_start(b, b)

    def group(g, carry):
        i = g * _NBUF
        for b in range(_NBUF):
            gather_wait(i + b, b)
            write_start(i + b, b)

        @pl.when(g + 1 < _NGROUP)
        def _():
            for b in range(_NBUF):
                write_wait(i + b, b)
                gather_start(i + _NBUF + b, b)

        return carry

    lax.fori_loop(0, _NGROUP, group, 0)
    for b in range(_NBUF):
        write_wait(_RB - _NBUF + b, b)


_CH = 16
_NCH = _RB // _CH
_CHL = _CH * _S * _D // 128


@functools.partial(
    pl.kernel,
    out_type=jax.ShapeDtypeStruct((_NB, _S, _D), jnp.float32),
    mesh=_mesh,
    scratch_types=[
        [pltpu.VMEM((_CHL, 128), jnp.float32) for _ in range(2)],
        [pltpu.SemaphoreType.DMA for _ in range(2)],
        [pltpu.SemaphoreType.DMA for _ in range(2)],
    ],
)
def _retile(flat_hbm, out_hbm, buf, rsem, wsem):
    wid = lax.axis_index("s") * 2 + lax.axis_index("c")
    i0 = wid * _RB
    l0 = wid * (_RB * _S * _D // 128)

    def rd_start(ch, b):
        pltpu.async_copy(
            flat_hbm.at[pl.ds(l0 + ch * _CHL, _CHL)], buf[b], rsem[b]
        )

    def rd_wait(ch, b):
        pltpu.make_async_copy(
            flat_hbm.at[pl.ds(l0 + ch * _CHL, _CHL)], buf[b], rsem[b]
        ).wait()

    def wr_start(ch, b):
        pltpu.async_copy(
            buf[b].reshape(_CH, _S, _D),
            out_hbm.at[pl.ds(i0 + ch * _CH, _CH)],
            wsem[b],
        )

    def wr_wait(ch, b):
        pltpu.make_async_copy(
            buf[b].reshape(_CH, _S, _D),
            out_hbm.at[pl.ds(i0 + ch * _CH, _CH)],
            wsem[b],
        ).wait()

    for b in range(2):
        rd_start(b, b)

    def pair(p, carry):
        ch0 = p * 2
        for b in range(2):
            rd_wait(ch0 + b, b)
            wr_start(ch0 + b, b)

        @pl.when(p + 1 < _NCH // 2)
        def _():
            for b in range(2):
                wr_wait(ch0 + b, b)
                rd_start(ch0 + 2 + b, b)

        return carry

    lax.fori_loop(0, _NCH // 2, pair, 0)
    for b in range(2):
        wr_wait(_NCH - 2 + b, b)


def kernel(input, embeddings):
    idxp = _prep(input.astype(jnp.int32))
    return _gather(idxp, embeddings)
